# Initial kernel scaffold; baseline (speedup 1.0000x reference)
import jax
import jax.numpy as jnp
from jax.experimental import pallas as pl


def kernel(x, edge_index_0, edge_index_1, edge_index_2, edge_index_3, edge_index_4, edge_index_5, edge_attr_0, edge_attr_1, edge_attr_2, Wg, bg, Wa1, ba1, Wa2, Wc, bc):
    raise NotImplementedError("write your pallas kernel here")


# speedup vs baseline: 7.4700x; 7.4700x over previous
import functools

import jax
import jax.numpy as jnp
from jax import lax
from jax.experimental import pallas as pl
from jax.experimental.pallas import tpu as pltpu
from jax.experimental.pallas import tpu_sc as plsc

N = 10000
E = 160000
D = 128
M = 6
NLAYER = 2
C = 8

NC = 2
NS = 16
CK = 128
CPT = 40
EP = NC * NS * CPT * CK
NA = N + 8
ROWS_A = 640
ROWS_B = N - 15 * ROWS_A

_f32 = jnp.float32
_i32 = jnp.int32


def _fill_vmem(ref, nrows, value):
    v = jnp.full((16,), value, _f32)

    def body(r, _):
        for f in range(8):
            ref[r, pl.ds(f * 16, 16)] = v
        return 0

    lax.fori_loop(0, nrows, body, 0)


def _fill_vmem_1d(ref, n16, value):
    v = jnp.full((16,), value, _f32)

    def body(r, _):
        ref[pl.ds(r * 16, 16)] = v
        return 0

    lax.fori_loop(0, n16, body, 0)


def _deg_body(e0, e1, e2, e3, e4, e5, w0, w1, w2,
              o0, o1, o2, o3, o4, o5,
              d0, d1, d2, d3, d4, d5, colv, wv, onesv, zv, dbuf):
    eis = [e0, e1, e2, e3, e4, e5]
    ws = [w0, w1, w2, None, None, None]
    degs = [d0, d1, d2, d3, d4, d5]
    outs = [o0, o1, o2, o3, o4, o5]
    core = lax.axis_index("c")
    tid = lax.axis_index("s")
    choff = core * (NS * CPT) + tid * CPT

    _fill_vmem_1d(onesv, CK // 16, 1.0)
    _fill_vmem_1d(zv, ROWS_A // 16, 0.0)

    for p in range(M):
        @pl.when(tid < NS - 1)
        def _(p=p):
            pltpu.sync_copy(zv, degs[p].at[pl.ds(tid * ROWS_A, ROWS_A)])

        @pl.when(tid == NS - 1)
        def _(p=p):
            pltpu.sync_copy(zv.at[pl.ds(0, ROWS_B)],
                            degs[p].at[pl.ds(15 * ROWS_A, ROWS_B)])
    plsc.subcore_barrier()

    for p in range(M):
        weighted = ws[p] is not None
        pltpu.sync_copy(eis[p].at[1, pl.ds(choff, CPT), :], colv)
        if weighted:
            pltpu.sync_copy(ws[p].at[pl.ds(choff, CPT), :], wv)

        def chunk(k, _, p=p, weighted=weighted):
            src = wv.at[k] if weighted else onesv
            pltpu.sync_copy(src, degs[p].at[colv.at[k]], add=True)
            return 0

        lax.fori_loop(0, CPT, chunk, 0)
    plsc.subcore_barrier()

    for p in range(M):
        @pl.when(tid < NS - 1)
        def _(p=p):
            pltpu.sync_copy(degs[p].at[pl.ds(tid * ROWS_A, ROWS_A)], dbuf)
            pltpu.sync_copy(dbuf,
                            outs[p].at[pl.ds(core * N + tid * ROWS_A, ROWS_A)])

        @pl.when(tid == NS - 1)
        def _(p=p):
            pltpu.sync_copy(degs[p].at[pl.ds(15 * ROWS_A, ROWS_B)],
                            dbuf.at[pl.ds(0, ROWS_B)])
            pltpu.sync_copy(dbuf.at[pl.ds(0, ROWS_B)],
                            outs[p].at[pl.ds(core * N + 15 * ROWS_A, ROWS_B)])


_deg_kernel = pl.kernel(
    _deg_body,
    out_type=tuple(jax.ShapeDtypeStruct((NC * N,), _f32) for _ in range(M)),
    mesh=plsc.VectorSubcoreMesh(core_axis_name="c", subcore_axis_name="s",
                                num_cores=NC, num_subcores=NS),
    scratch_types=(
        [pltpu.VMEM_SHARED((NA,), _f32) for _ in range(M)]
        + [pltpu.VMEM((CPT, CK), _i32),
           pltpu.VMEM((CPT, CK), _f32),
           pltpu.VMEM((CK,), _f32),
           pltpu.VMEM((ROWS_A,), _f32),
           pltpu.VMEM((ROWS_A,), _f32)]
    ),
)


def _esum_body(y0, y1, y2, y3, y4, y5, e0, e1, e2, e3, e4, e5, w0, w1, w2,
               out, acc, rowv, colv, wv, rows_v, zbuf, gsem):
    ys = [y0, y1, y2, y3, y4, y5]
    eis = [e0, e1, e2, e3, e4, e5]
    ws = [w0, w1, w2, None, None, None]
    core = lax.axis_index("c")
    tid = lax.axis_index("s")
    choff = core * (NS * CPT) + tid * CPT

    _fill_vmem(zbuf, CK, 0.0)

    def zero_own():
        @pl.when(tid < NS - 1)
        def _():
            for b in range(ROWS_A // CK):
                pltpu.sync_copy(
                    zbuf, acc.at[pl.ds(tid * ROWS_A + b * CK, CK), :])

        @pl.when(tid == NS - 1)
        def _():
            for b in range(ROWS_B // CK):
                pltpu.sync_copy(
                    zbuf, acc.at[pl.ds(15 * ROWS_A + b * CK, CK), :])
            rem = ROWS_B % CK
            if rem:
                pltpu.sync_copy(
                    zbuf.at[pl.ds(0, rem), :],
                    acc.at[pl.ds(15 * ROWS_A + (ROWS_B // CK) * CK, rem), :])

    def write_own(p):
        @pl.when(tid < NS - 1)
        def _():
            for b in range(ROWS_A // CK):
                off = tid * ROWS_A + b * CK
                pltpu.sync_copy(acc.at[pl.ds(off, CK), :], rows_v)
                pltpu.sync_copy(rows_v, out.at[core, p, pl.ds(off, CK), :])

        @pl.when(tid == NS - 1)
        def _():
            for b in range(ROWS_B // CK):
                off = 15 * ROWS_A + b * CK
                pltpu.sync_copy(acc.at[pl.ds(off, CK), :], rows_v)
                pltpu.sync_copy(rows_v, out.at[core, p, pl.ds(off, CK), :])
            rem = ROWS_B % CK
            if rem:
                off = 15 * ROWS_A + (ROWS_B // CK) * CK
                pltpu.sync_copy(acc.at[pl.ds(off, rem), :],
                                rows_v.at[pl.ds(0, rem), :])
                pltpu.sync_copy(rows_v.at[pl.ds(0, rem), :],
                                out.at[core, p, pl.ds(off, rem), :])

    zero_own()
    plsc.subcore_barrier()

    for p in range(M):
        weighted = ws[p] is not None
        pltpu.sync_copy(eis[p].at[0, pl.ds(choff, CPT), :], rowv)
        pltpu.sync_copy(eis[p].at[1, pl.ds(choff, CPT), :], colv)
        if weighted:
            pltpu.sync_copy(ws[p].at[pl.ds(choff, CPT), :], wv)

        def chunk(k, _, p=p, weighted=weighted):
            pltpu.async_copy(ys[p].at[rowv.at[k]], rows_v, gsem).wait()
            if weighted:
                def scale(g, _):
                    w16 = wv[k, pl.ds(g * 16, 16)]
                    for j in range(16):
                        wb = lax.gather(
                            w16, jnp.full((16, 1), j, _i32),
                            lax.GatherDimensionNumbers(
                                offset_dims=(), collapsed_slice_dims=(0,),
                                start_index_map=(0,)),
                            slice_sizes=(1,),
                            mode=lax.GatherScatterMode.PROMISE_IN_BOUNDS)
                        e = g * 16 + j
                        for f in range(8):
                            rows_v[e, pl.ds(f * 16, 16)] = (
                                rows_v[e, pl.ds(f * 16, 16)] * wb)
                    return 0
                lax.fori_loop(0, 8, scale, 0)
            pltpu.sync_copy(rows_v, acc.at[colv.at[k]], add=True)
            return 0

        lax.fori_loop(0, CPT, chunk, 0)
        plsc.subcore_barrier()
        write_own(p)
        if p < M - 1:
            zero_own()
        plsc.subcore_barrier()


_esum_kernel = pl.kernel(
    _esum_body,
    out_type=jax.ShapeDtypeStruct((NC, M, N, D), _f32),
    mesh=plsc.VectorSubcoreMesh(core_axis_name="c", subcore_axis_name="s",
                                num_cores=NC, num_subcores=NS),
    scratch_types=[
        pltpu.VMEM_SHARED((NA, D), _f32),
        pltpu.VMEM((CPT, CK), _i32),
        pltpu.VMEM((CPT, CK), _i32),
        pltpu.VMEM((CPT, CK), _f32),
        pltpu.VMEM((CK, D), _f32),
        pltpu.VMEM((CK, D), _f32),
        pltpu.SemaphoreType.DMA,
    ],
)


BN = 1024
NB = (N + BN - 1) // BN


def _mm_body(h_ref, wg_ref, degp_ref, *out_refs):
    h = h_ref[...]
    dp = degp_ref[...]
    for i in range(M):
        dinv = lax.rsqrt(dp[0, i] + dp[1, i] + 1.0)
        xw = jnp.dot(h, wg_ref[i], preferred_element_type=_f32)
        out_refs[i][...] = xw * dinv[:, None]


def _mm_call(h, wg_l, degp):
    return pl.pallas_call(
        _mm_body,
        grid=(NB,),
        in_specs=[
            pl.BlockSpec((BN, D), lambda i: (i, 0)),
            pl.BlockSpec((M, D, D), lambda i: (0, 0, 0)),
            pl.BlockSpec((NC, M, BN), lambda i: (0, 0, i)),
        ],
        out_specs=[pl.BlockSpec((BN, D), lambda i: (i, 0))] * M,
        out_shape=[jax.ShapeDtypeStruct((N, D), _f32)] * M,
    )(h, wg_l, degp)


def _combine_body(s_ref, y0, y1, y2, y3, y4, y5, degp_ref, bg_ref,
                  wa1_ref, ba1_ref, wa2_ref, wc_ref, bc_ref, out_ref,
                  *, emit_logits):
    yrefs = [y0, y1, y2, y3, y4, y5]
    dp = degp_ref[...]
    wa1 = wa1_ref[...]
    ba1 = ba1_ref[...]
    wa2 = wa2_ref[...]
    zs, ss = [], []
    for i in range(M):
        dinv = lax.rsqrt(dp[0, i] + dp[1, i] + 1.0)
        z = (s_ref[0, i] + s_ref[1, i] + yrefs[i][...]) * dinv[:, None]
        z = z + bg_ref[i][None, :]
        t = jnp.tanh(jnp.dot(z, wa1, preferred_element_type=_f32) + ba1)
        s = jnp.sum(t * wa2, axis=1, keepdims=True)
        zs.append(z)
        ss.append(s)
    m = ss[0]
    for i in range(1, M):
        m = jnp.maximum(m, ss[i])
    es = [jnp.exp(s - m) for s in ss]
    den = es[0]
    for i in range(1, M):
        den = den + es[i]
    h = es[0] * zs[0]
    for i in range(1, M):
        h = h + es[i] * zs[i]
    h = h / den
    if emit_logits:
        out_ref[...] = jnp.dot(h, wc_ref[...], preferred_element_type=_f32) \
            + bc_ref[...]
    else:
        out_ref[...] = h


def _combine_call(s, ys, degp, bg_l, wa1_l, ba1_l, wa2_l, wc, bc,
                  emit_logits):
    oc = C if emit_logits else D
    return pl.pallas_call(
        functools.partial(_combine_body, emit_logits=emit_logits),
        grid=(NB,),
        in_specs=[
            pl.BlockSpec((NC, M, BN, D), lambda i: (0, 0, i, 0)),
        ] + [pl.BlockSpec((BN, D), lambda i: (i, 0))] * M + [
            pl.BlockSpec((NC, M, BN), lambda i: (0, 0, i)),
            pl.BlockSpec((M, D), lambda i: (0, 0)),
            pl.BlockSpec((D, D), lambda i: (0, 0)),
            pl.BlockSpec((1, D), lambda i: (0, 0)),
            pl.BlockSpec((1, D), lambda i: (0, 0)),
            pl.BlockSpec((D, C), lambda i: (0, 0)),
            pl.BlockSpec((1, C), lambda i: (0, 0)),
        ],
        out_specs=pl.BlockSpec((BN, oc), lambda i: (i, 0)),
        out_shape=jax.ShapeDtypeStruct((N, oc), _f32),
    )(s, *ys, degp, bg_l, wa1_l, ba1_l, wa2_l, wc, bc)


def kernel(x, edge_index_0, edge_index_1, edge_index_2, edge_index_3,
           edge_index_4, edge_index_5, edge_attr_0, edge_attr_1, edge_attr_2,
           Wg, bg, Wa1, ba1, Wa2, Wc, bc):
    pad = EP - E
    eis3 = []
    for e in (edge_index_0, edge_index_1, edge_index_2, edge_index_3,
              edge_index_4, edge_index_5):
        rows = jnp.concatenate([e[0], jnp.zeros((pad,), _i32)])
        cols = jnp.concatenate([e[1], jnp.full((pad,), N, _i32)])
        eis3.append(jnp.stack([rows, cols]).reshape(2, EP // CK, CK))
    ws3 = [jnp.concatenate([w, jnp.zeros((pad,), _f32)]).reshape(EP // CK, CK)
           for w in (edge_attr_0, edge_attr_1, edge_attr_2)]

    degs = _deg_kernel(*eis3, *ws3)
    degp = jnp.stack([d.reshape(NC, N) for d in degs], axis=1)

    h = x
    for l in range(NLAYER):
        ys = _mm_call(h, Wg[l], degp)
        s = _esum_kernel(*ys, *eis3, *ws3)
        h = _combine_call(
            s, ys, degp, bg[l], Wa1[l], ba1[l].reshape(1, D),
            Wa2[l, :, 0].reshape(1, D), Wc, bc.reshape(1, C),
            emit_logits=(l == NLAYER - 1))
    return h


# speedup vs baseline: 8.2561x; 1.1052x over previous
import functools

import jax
import jax.numpy as jnp
from jax import lax
from jax.experimental import pallas as pl
from jax.experimental.pallas import tpu as pltpu
from jax.experimental.pallas import tpu_sc as plsc

N = 10000
E = 160000
D = 128
M = 6
NLAYER = 2
C = 8

NC = 2
NS = 16
CK = 128
CPT = 40
EP = NC * NS * CPT * CK
NA = N + 8
ROWS_A = 640
ROWS_B = N - 15 * ROWS_A

_f32 = jnp.float32
_i32 = jnp.int32


def _fill_vmem(ref, nrows, value):
    v = jnp.full((16,), value, _f32)

    def body(r, _):
        for f in range(8):
            ref[r, pl.ds(f * 16, 16)] = v
        return 0

    lax.fori_loop(0, nrows, body, 0)


def _fill_vmem_1d(ref, n16, value):
    v = jnp.full((16,), value, _f32)

    def body(r, _):
        ref[pl.ds(r * 16, 16)] = v
        return 0

    lax.fori_loop(0, n16, body, 0)


def _deg_body(e0, e1, e2, e3, e4, e5, w0, w1, w2,
              o0, o1, o2, o3, o4, o5,
              d0, d1, d2, d3, d4, d5, colv, wv, onesv, zv, dbuf, sem):
    eis = [e0, e1, e2, e3, e4, e5]
    ws = [w0, w1, w2, None, None, None]
    degs = [d0, d1, d2, d3, d4, d5]
    outs = [o0, o1, o2, o3, o4, o5]
    core = lax.axis_index("c")
    tid = lax.axis_index("s")
    choff = core * (NS * CPT) + tid * CPT

    _fill_vmem_1d(onesv, CK // 16, 1.0)
    _fill_vmem_1d(zv, ROWS_A // 16, 0.0)

    for p in range(M):
        @pl.when(tid < NS - 1)
        def _(p=p):
            pltpu.sync_copy(zv, degs[p].at[pl.ds(tid * ROWS_A, ROWS_A)])

        @pl.when(tid == NS - 1)
        def _(p=p):
            pltpu.sync_copy(zv.at[pl.ds(0, ROWS_B)],
                            degs[p].at[pl.ds(15 * ROWS_A, ROWS_B)])
    plsc.subcore_barrier()

    stages = []
    for p in range(M):
        stages.append(pltpu.async_copy(
            eis[p].at[1, pl.ds(choff, CPT), :],
            colv.at[pl.ds(p * CPT, CPT)], sem))
        if ws[p] is not None:
            stages.append(pltpu.async_copy(
                ws[p].at[pl.ds(choff, CPT), :],
                wv.at[pl.ds(p * CPT, CPT)], sem))
    for st in stages:
        st.wait()

    adds = []
    for p in range(M):
        weighted = ws[p] is not None

        def chunk(k, _, p=p, weighted=weighted):
            src = wv.at[p * CPT + k] if weighted else onesv
            pltpu.async_copy(src, degs[p].at[colv.at[p * CPT + k]], sem,
                             add=True)
            return 0

        lax.fori_loop(0, CPT, chunk, 0)

        def drain(k, _, p=p, weighted=weighted):
            src = wv.at[p * CPT + k] if weighted else onesv
            pltpu.make_async_copy(
                src, degs[p].at[colv.at[p * CPT + k]], sem).wait()
            return 0

        lax.fori_loop(0, CPT, drain, 0)
    plsc.subcore_barrier()

    for p in range(M):
        @pl.when(tid < NS - 1)
        def _(p=p):
            pltpu.sync_copy(degs[p].at[pl.ds(tid * ROWS_A, ROWS_A)], dbuf)
            pltpu.sync_copy(dbuf,
                            outs[p].at[pl.ds(core * N + tid * ROWS_A, ROWS_A)])

        @pl.when(tid == NS - 1)
        def _(p=p):
            pltpu.sync_copy(degs[p].at[pl.ds(15 * ROWS_A, ROWS_B)],
                            dbuf.at[pl.ds(0, ROWS_B)])
            pltpu.sync_copy(dbuf.at[pl.ds(0, ROWS_B)],
                            outs[p].at[pl.ds(core * N + 15 * ROWS_A, ROWS_B)])


_deg_kernel = pl.kernel(
    _deg_body,
    out_type=tuple(jax.ShapeDtypeStruct((NC * N,), _f32) for _ in range(M)),
    mesh=plsc.VectorSubcoreMesh(core_axis_name="c", subcore_axis_name="s",
                                num_cores=NC, num_subcores=NS),
    scratch_types=(
        [pltpu.VMEM_SHARED((NA,), _f32) for _ in range(M)]
        + [pltpu.VMEM((M * CPT, CK), _i32),
           pltpu.VMEM((3 * CPT, CK), _f32),
           pltpu.VMEM((CK,), _f32),
           pltpu.VMEM((ROWS_A,), _f32),
           pltpu.VMEM((ROWS_A,), _f32),
           pltpu.SemaphoreType.DMA]
    ),
)


def _esum_body(y0, y1, y2, y3, y4, y5, e0, e1, e2, e3, e4, e5, w0, w1, w2,
               out, acc, rowv, colv, wv, rows_v, rows_v2,
               gs0, gs1, ss0, ss1):
    ys = [y0, y1, y2, y3, y4, y5]
    eis = [e0, e1, e2, e3, e4, e5]
    ws = [w0, w1, w2, None, None, None]
    core = lax.axis_index("c")
    tid = lax.axis_index("s")
    choff = core * (NS * CPT) + tid * CPT

    def zero_own():
        _fill_vmem(rows_v, CK, 0.0)

        @pl.when(tid < NS - 1)
        def _():
            for b in range(ROWS_A // CK):
                pltpu.sync_copy(
                    rows_v, acc.at[pl.ds(tid * ROWS_A + b * CK, CK), :])

        @pl.when(tid == NS - 1)
        def _():
            for b in range(ROWS_B // CK):
                pltpu.sync_copy(
                    rows_v, acc.at[pl.ds(15 * ROWS_A + b * CK, CK), :])
            rem = ROWS_B % CK
            if rem:
                pltpu.sync_copy(
                    rows_v.at[pl.ds(0, rem), :],
                    acc.at[pl.ds(15 * ROWS_A + (ROWS_B // CK) * CK, rem), :])

    def write_own(p):
        @pl.when(tid < NS - 1)
        def _():
            for b in range(ROWS_A // CK):
                off = tid * ROWS_A + b * CK
                pltpu.sync_copy(acc.at[pl.ds(off, CK), :], rows_v2)
                pltpu.sync_copy(rows_v2, out.at[core, p, pl.ds(off, CK), :])

        @pl.when(tid == NS - 1)
        def _():
            for b in range(ROWS_B // CK):
                off = 15 * ROWS_A + b * CK
                pltpu.sync_copy(acc.at[pl.ds(off, CK), :], rows_v2)
                pltpu.sync_copy(rows_v2, out.at[core, p, pl.ds(off, CK), :])
            rem = ROWS_B % CK
            if rem:
                off = 15 * ROWS_A + (ROWS_B // CK) * CK
                pltpu.sync_copy(acc.at[pl.ds(off, rem), :],
                                rows_v2.at[pl.ds(0, rem), :])
                pltpu.sync_copy(rows_v2.at[pl.ds(0, rem), :],
                                out.at[core, p, pl.ds(off, rem), :])

    zero_own()
    plsc.subcore_barrier()

    for p in range(M):
        weighted = ws[p] is not None
        pltpu.sync_copy(eis[p].at[0, pl.ds(choff, CPT), :], rowv)
        pltpu.sync_copy(eis[p].at[1, pl.ds(choff, CPT), :], colv)
        if weighted:
            pltpu.sync_copy(ws[p].at[pl.ds(choff, CPT), :], wv)

        def g_start(k, buf, sem, p=p):
            pltpu.async_copy(ys[p].at[rowv.at[k]], buf, sem)

        def g_wait(buf, sem, p=p):
            pltpu.make_async_copy(ys[p].at[rowv.at[0]], buf, sem).wait()

        def s_start(k, buf, sem):
            pltpu.async_copy(buf, acc.at[colv.at[k]], sem, add=True)

        def s_wait(buf, sem):
            pltpu.make_async_copy(buf, acc.at[colv.at[0]], sem).wait()

        def scale(k, buf):
            def grp(g, _):
                w16 = wv[k, pl.ds(g * 16, 16)]
                for j in range(16):
                    wb = lax.gather(
                        w16, jnp.full((16, 1), j, _i32),
                        lax.GatherDimensionNumbers(
                            offset_dims=(), collapsed_slice_dims=(0,),
                            start_index_map=(0,)),
                        slice_sizes=(1,),
                        mode=lax.GatherScatterMode.PROMISE_IN_BOUNDS)
                    e = g * 16 + j
                    for f in range(8):
                        buf[e, pl.ds(f * 16, 16)] = (
                            buf[e, pl.ds(f * 16, 16)] * wb)
                return 0
            lax.fori_loop(0, 8, grp, 0)

        g_start(0, rows_v, gs0)

        def body(i, _, weighted=weighted):
            k0 = 2 * i
            k1 = 2 * i + 1
            g_wait(rows_v, gs0)
            g_start(k1, rows_v2, gs1)
            if weighted:
                scale(k0, rows_v)
            s_start(k0, rows_v, ss0)
            g_wait(rows_v2, gs1)
            s_wait(rows_v, ss0)
            kn = jnp.minimum(k0 + 2, CPT - 1)
            g_start(kn, rows_v, gs0)
            if weighted:
                scale(k1, rows_v2)
            s_start(k1, rows_v2, ss1)
            s_wait(rows_v2, ss1)
            return 0

        lax.fori_loop(0, CPT // 2, body, 0)
        g_wait(rows_v, gs0)
        plsc.subcore_barrier()
        write_own(p)
        if p < M - 1:
            zero_own()
        plsc.subcore_barrier()


_esum_kernel = pl.kernel(
    _esum_body,
    out_type=jax.ShapeDtypeStruct((NC, M, N, D), _f32),
    mesh=plsc.VectorSubcoreMesh(core_axis_name="c", subcore_axis_name="s",
                                num_cores=NC, num_subcores=NS),
    scratch_types=[
        pltpu.VMEM_SHARED((NA, D), _f32),
        pltpu.VMEM((CPT, CK), _i32),
        pltpu.VMEM((CPT, CK), _i32),
        pltpu.VMEM((CPT, CK), _f32),
        pltpu.VMEM((CK, D), _f32),
        pltpu.VMEM((CK, D), _f32),
        pltpu.SemaphoreType.DMA,
        pltpu.SemaphoreType.DMA,
        pltpu.SemaphoreType.DMA,
        pltpu.SemaphoreType.DMA,
    ],
)


BN = 1024
NB = (N + BN - 1) // BN


def _mm_body(h_ref, wg_ref, degp_ref, *out_refs):
    h = h_ref[...]
    dp = degp_ref[...]
    for i in range(M):
        dinv = lax.rsqrt(dp[0, i] + dp[1, i] + 1.0)
        xw = jnp.dot(h, wg_ref[i], preferred_element_type=_f32)
        out_refs[i][...] = xw * dinv[:, None]


def _mm_call(h, wg_l, degp):
    return pl.pallas_call(
        _mm_body,
        grid=(NB,),
        in_specs=[
            pl.BlockSpec((BN, D), lambda i: (i, 0)),
            pl.BlockSpec((M, D, D), lambda i: (0, 0, 0)),
            pl.BlockSpec((NC, M, BN), lambda i: (0, 0, i)),
        ],
        out_specs=[pl.BlockSpec((BN, D), lambda i: (i, 0))] * M,
        out_shape=[jax.ShapeDtypeStruct((N, D), _f32)] * M,
    )(h, wg_l, degp)


def _combine_body(s_ref, y0, y1, y2, y3, y4, y5, degp_ref, bg_ref,
                  wa1_ref, ba1_ref, wa2_ref, wc_ref, bc_ref, out_ref,
                  *, emit_logits):
    yrefs = [y0, y1, y2, y3, y4, y5]
    dp = degp_ref[...]
    wa1 = wa1_ref[...]
    ba1 = ba1_ref[...]
    wa2 = wa2_ref[...]
    zs, ss = [], []
    for i in range(M):
        dinv = lax.rsqrt(dp[0, i] + dp[1, i] + 1.0)
        z = (s_ref[0, i] + s_ref[1, i] + yrefs[i][...]) * dinv[:, None]
        z = z + bg_ref[i][None, :]
        t = jnp.tanh(jnp.dot(z, wa1, preferred_element_type=_f32) + ba1)
        s = jnp.sum(t * wa2, axis=1, keepdims=True)
        zs.append(z)
        ss.append(s)
    m = ss[0]
    for i in range(1, M):
        m = jnp.maximum(m, ss[i])
    es = [jnp.exp(s - m) for s in ss]
    den = es[0]
    for i in range(1, M):
        den = den + es[i]
    h = es[0] * zs[0]
    for i in range(1, M):
        h = h + es[i] * zs[i]
    h = h / den
    if emit_logits:
        out_ref[...] = jnp.dot(h, wc_ref[...], preferred_element_type=_f32) \
            + bc_ref[...]
    else:
        out_ref[...] = h


def _combine_call(s, ys, degp, bg_l, wa1_l, ba1_l, wa2_l, wc, bc,
                  emit_logits):
    oc = C if emit_logits else D
    return pl.pallas_call(
        functools.partial(_combine_body, emit_logits=emit_logits),
        grid=(NB,),
        in_specs=[
            pl.BlockSpec((NC, M, BN, D), lambda i: (0, 0, i, 0)),
        ] + [pl.BlockSpec((BN, D), lambda i: (i, 0))] * M + [
            pl.BlockSpec((NC, M, BN), lambda i: (0, 0, i)),
            pl.BlockSpec((M, D), lambda i: (0, 0)),
            pl.BlockSpec((D, D), lambda i: (0, 0)),
            pl.BlockSpec((1, D), lambda i: (0, 0)),
            pl.BlockSpec((1, D), lambda i: (0, 0)),
            pl.BlockSpec((D, C), lambda i: (0, 0)),
            pl.BlockSpec((1, C), lambda i: (0, 0)),
        ],
        out_specs=pl.BlockSpec((BN, oc), lambda i: (i, 0)),
        out_shape=jax.ShapeDtypeStruct((N, oc), _f32),
    )(s, *ys, degp, bg_l, wa1_l, ba1_l, wa2_l, wc, bc)


def kernel(x, edge_index_0, edge_index_1, edge_index_2, edge_index_3,
           edge_index_4, edge_index_5, edge_attr_0, edge_attr_1, edge_attr_2,
           Wg, bg, Wa1, ba1, Wa2, Wc, bc):
    pad = EP - E
    eis3 = []
    for e in (edge_index_0, edge_index_1, edge_index_2, edge_index_3,
              edge_index_4, edge_index_5):
        rows = jnp.concatenate([e[0], jnp.zeros((pad,), _i32)])
        cols = jnp.concatenate([e[1], jnp.full((pad,), N, _i32)])
        eis3.append(jnp.stack([rows, cols]).reshape(2, EP // CK, CK))
    ws3 = [jnp.concatenate([w, jnp.zeros((pad,), _f32)]).reshape(EP // CK, CK)
           for w in (edge_attr_0, edge_attr_1, edge_attr_2)]

    degs = _deg_kernel(*eis3, *ws3)
    degp = jnp.stack([d.reshape(NC, N) for d in degs], axis=1)

    h = x
    for l in range(NLAYER):
        ys = _mm_call(h, Wg[l], degp)
        s = _esum_kernel(*ys, *eis3, *ws3)
        h = _combine_call(
            s, ys, degp, bg[l], Wa1[l], ba1[l].reshape(1, D),
            Wa2[l, :, 0].reshape(1, D), Wc, bc.reshape(1, C),
            emit_logits=(l == NLAYER - 1))
    return h
